# baseline (speedup 1.0000x reference)
import functools

import jax
import jax.numpy as jnp
from jax import lax
from jax.experimental import pallas as pl
from jax.experimental.pallas import tpu as pltpu
from jax.experimental.pallas import tpu_sc as plsc

N = 10000
E = 320000
NPAD = 10240
ROWS_PER_TILE = 640
CHW = 128
FC = 126
SCC = 36
TOTCH = 16 * (FC + SCC)
EPAD = TOTCH * CHW
RD = 9
RL = 2
EPS = 1e-5

_mesh = plsc.VectorSubcoreMesh(core_axis_name="c", subcore_axis_name="s")
_sc_params = pltpu.CompilerParams(use_tc_tiling_on_sc=False)


def _fill_rows(buf, val, d):
    v16 = jnp.full((16,), val, jnp.float32)

    def body(i, carry):
        for cc in range(d // 16):
            buf[i, pl.ds(cc * 16, 16)] = v16
        return carry

    lax.fori_loop(0, CHW, body, 0)


def _load_idx(c, s, idx_hbm, idx_v):
    @pl.when(c == 0)
    def _():
        pltpu.sync_copy(idx_hbm.at[pl.ds(s * FC, FC)], idx_v)

    @pl.when(c == 1)
    def _():
        pltpu.sync_copy(idx_hbm.at[pl.ds(16 * FC + s * SCC, SCC)],
                        idx_v.at[pl.ds(0, SCC)])


@functools.partial(
    pl.kernel,
    out_type=jax.ShapeDtypeStruct((2, NPAD, 16), jnp.float32),
    mesh=_mesh,
    compiler_params=_sc_params,
    scratch_types=[
        pltpu.VMEM((FC, CHW), jnp.int32),
        pltpu.VMEM((CHW, 16), jnp.float32),
        pltpu.VMEM_SHARED((NPAD, 16), jnp.float32),
        pltpu.SemaphoreType.DMA,
    ],
)
def _sc_degree(dst_hbm, out_hbm, didx, buf, degsh, sem):
    c = lax.axis_index("c")
    s = lax.axis_index("s")
    nch = jnp.where(c == 0, FC, SCC)

    _load_idx(c, s, dst_hbm, didx)
    _fill_rows(buf, 0.0, 16)
    for k in range(ROWS_PER_TILE // CHW):
        pltpu.sync_copy(buf, degsh.at[pl.ds(s * ROWS_PER_TILE + k * CHW, CHW)])
    _fill_rows(buf, 1.0, 16)
    plsc.subcore_barrier()

    def fire(j, carry):
        pltpu.async_copy(buf, degsh.at[didx.at[j]], sem, add=True)
        return carry

    def drain(j, carry):
        pltpu.make_async_copy(buf, degsh.at[didx.at[j]], sem).wait()
        return carry

    lax.fori_loop(0, nch, fire, 0)
    lax.fori_loop(0, nch, drain, 0)
    plsc.subcore_barrier()
    pltpu.sync_copy(
        degsh.at[pl.ds(s * ROWS_PER_TILE, ROWS_PER_TILE)],
        out_hbm.at[c].at[pl.ds(s * ROWS_PER_TILE, ROWS_PER_TILE)],
    )


def _make_sc_agg(d):
    @functools.partial(
        pl.kernel,
        out_type=jax.ShapeDtypeStruct((2, NPAD, d), jnp.float32),
        mesh=_mesh,
        compiler_params=_sc_params,
        scratch_types=[
            pltpu.VMEM((3, RD, CHW), jnp.int32),
            pltpu.VMEM((3, RD, CHW), jnp.int32),
            pltpu.VMEM((RD, CHW, d), jnp.float32),
            pltpu.VMEM_SHARED((NPAD, d), jnp.float32),
            pltpu.SemaphoreType.DMA,
            pltpu.SemaphoreType.DMA,
            pltpu.SemaphoreType.DMA,
        ],
    )
    def sc_agg(hp_hbm, src_hbm, dst_hbm, out_hbm, sidx, didx, rows,
               aggsh, gsem, ssem, isem):
        c = lax.axis_index("c")
        s = lax.axis_index("s")
        nch = jnp.where(c == 0, FC, SCC)
        ntrip = jnp.where(c == 0, FC // RD, SCC // RD)
        cbase = jnp.where(c == 0, s * FC, 16 * FC + s * SCC)

        def fire_idx(m, p):
            pltpu.async_copy(src_hbm.at[pl.ds(cbase + m * RD, RD)],
                             sidx.at[p], isem)
            pltpu.async_copy(dst_hbm.at[pl.ds(cbase + m * RD, RD)],
                             didx.at[p], isem)

        def wait_idx(p):
            pltpu.make_async_copy(src_hbm.at[pl.ds(cbase, RD)],
                                  sidx.at[p], isem).wait()
            pltpu.make_async_copy(dst_hbm.at[pl.ds(cbase, RD)],
                                  didx.at[p], isem).wait()

        pltpu.sync_copy(src_hbm.at[pl.ds(cbase, RD)], sidx.at[0])
        pltpu.sync_copy(dst_hbm.at[pl.ds(cbase, RD)], didx.at[0])

        _fill_rows(rows.at[0], 0.0, d)
        for k in range(ROWS_PER_TILE // CHW):
            pltpu.sync_copy(
                rows.at[0], aggsh.at[pl.ds(s * ROWS_PER_TILE + k * CHW, CHW)])
        plsc.subcore_barrier()

        def fire_g(b, p, r):
            pltpu.async_copy(hp_hbm.at[sidx.at[p].at[r]], rows.at[b], gsem)

        def wait_g(b):
            pltpu.make_async_copy(
                hp_hbm.at[sidx.at[0].at[0]], rows.at[b], gsem).wait()

        def fire_s(b, p, r):
            pltpu.async_copy(rows.at[b], aggsh.at[didx.at[p].at[r]], ssem,
                             add=True)

        def wait_s(b):
            pltpu.make_async_copy(
                rows.at[b], aggsh.at[didx.at[0].at[0]], ssem).wait()

        for j in range(RD - RL):
            fire_g(j, 0, j)

        @pl.when(1 < ntrip)
        def _():
            fire_idx(1, 1)

        def body(t2, carry):
            q = lax.rem(t2, 3)
            qn = lax.rem(t2 + 1, 3)
            for b in range(RD):
                t = t2 * RD + b

                @pl.when(t >= RL)
                def _():
                    wait_s((t - RL) % RD)

                if b == RL:
                    @pl.when(t2 + 1 < ntrip)
                    def _():
                        wait_idx(qn)

                    @pl.when(t2 + 2 < ntrip)
                    def _():
                        fire_idx(t2 + 2, lax.rem(t2 + 2, 3))

                if b < RL:
                    @pl.when(t < nch - RD + RL)
                    def _():
                        fire_g((t - RL) % RD, q, RD - RL + b)
                else:
                    @pl.when(t < nch - RD + RL)
                    def _():
                        fire_g((t - RL) % RD, qn, b - RL)

                wait_g(b)
                fire_s(b, q, b)
            return carry

        lax.fori_loop(0, ntrip, body, 0)
        for i in range(RL):
            wait_s(RD - RL + i)
        plsc.subcore_barrier()
        pltpu.sync_copy(
            aggsh.at[pl.ds(s * ROWS_PER_TILE, ROWS_PER_TILE)],
            out_hbm.at[c].at[pl.ds(s * ROWS_PER_TILE, ROWS_PER_TILE)],
        )

    return sc_agg


_sc_agg64 = _make_sc_agg(64)
_sc_agg16 = _make_sc_agg(16)


def _dinv_from(degp_ref):
    deg = degp_ref[0, :N, 0:1] + degp_ref[1, :N, 0:1] + 1.0
    return lax.rsqrt(deg)


def _tc_mm(x, w1):
    def body(x_ref, w_ref, out_ref):
        out_ref[...] = jnp.dot(x_ref[...], w_ref[...],
                               preferred_element_type=jnp.float32)

    return pl.pallas_call(
        body, out_shape=jax.ShapeDtypeStruct((N, w1.shape[1]), jnp.float32),
    )(x, w1)


def _tc_scale(xw, degp):
    def body(xw_ref, degp_ref, out_ref):
        out_ref[...] = xw_ref[...] * _dinv_from(degp_ref)

    return pl.pallas_call(
        body, out_shape=jax.ShapeDtypeStruct(xw.shape, jnp.float32),
    )(xw, degp)


def _tc_mid(aggp, hp, degp, g, be, wn):
    def body(aggp_ref, hp_ref, degp_ref, g_ref, be_ref, w_ref, out_ref):
        dinv = _dinv_from(degp_ref)
        conv = dinv * (aggp_ref[0, :N] + aggp_ref[1, :N] + hp_ref[...])
        mu = jnp.mean(conv, axis=0, keepdims=True)
        xc = conv - mu
        var = jnp.mean(xc * xc, axis=0, keepdims=True)
        h = g_ref[...] * xc * lax.rsqrt(var + EPS) + be_ref[...]
        h = jnp.maximum(h, 0.0) * dinv
        out_ref[...] = jnp.dot(h, w_ref[...],
                               preferred_element_type=jnp.float32)

    return pl.pallas_call(
        body, out_shape=jax.ShapeDtypeStruct((N, wn.shape[1]), jnp.float32),
    )(aggp, hp, degp, g, be, wn)


def _tc_post(aggp, hp, degp, g, be):
    def body(aggp_ref, hp_ref, degp_ref, g_ref, be_ref, out_ref):
        dinv = _dinv_from(degp_ref)
        conv = dinv * (aggp_ref[0, :N] + aggp_ref[1, :N] + hp_ref[...])
        mu = jnp.mean(conv, axis=0, keepdims=True)
        xc = conv - mu
        var = jnp.mean(xc * xc, axis=0, keepdims=True)
        out_ref[...] = g_ref[...] * xc * lax.rsqrt(var + EPS) + be_ref[...]

    return pl.pallas_call(
        body, out_shape=jax.ShapeDtypeStruct((N, 16), jnp.float32),
    )(aggp, hp, degp, g, be)


def kernel(x, edge_index, W1, b1, g1, be1, W2, b2, g2, be2, W3, b3, g3, be3):
    del b1, b2, b3
    src = edge_index[0].astype(jnp.int32)
    dst = edge_index[1].astype(jnp.int32)
    pad = EPAD - E
    src_t = jnp.concatenate([src, jnp.zeros((pad,), jnp.int32)]
                            ).reshape(TOTCH, CHW)
    dst_t = jnp.concatenate([dst, jnp.full((pad,), N, jnp.int32)]
                            ).reshape(TOTCH, CHW)

    degp = _sc_degree(dst_t)

    g1r = g1.reshape(1, -1)
    be1r = be1.reshape(1, -1)
    g2r = g2.reshape(1, -1)
    be2r = be2.reshape(1, -1)
    g3p = jnp.concatenate([g3, jnp.ones((16 - g3.shape[0],), jnp.float32)]
                          ).reshape(1, 16)
    be3p = jnp.concatenate([be3, jnp.zeros((16 - be3.shape[0],), jnp.float32)]
                           ).reshape(1, 16)
    w3p = jnp.concatenate(
        [W3, jnp.zeros((W3.shape[0], 16 - W3.shape[1]), jnp.float32)], axis=1)

    hp1 = _tc_scale(_tc_mm(x, W1), degp)
    agg1 = _sc_agg64(hp1, src_t, dst_t)
    hp2 = _tc_mid(agg1, hp1, degp, g1r, be1r, W2)
    agg2 = _sc_agg64(hp2, src_t, dst_t)
    hp3 = _tc_mid(agg2, hp2, degp, g2r, be2r, w3p)
    agg3 = _sc_agg16(hp3, src_t, dst_t)
    out = _tc_post(agg3, hp3, degp, g3p, be3p)
    return out[:, :W3.shape[1]]


# speedup vs baseline: 1.4374x; 1.4374x over previous
import functools

import jax
import jax.numpy as jnp
from jax import lax
from jax.experimental import pallas as pl
from jax.experimental.pallas import tpu as pltpu
from jax.experimental.pallas import tpu_sc as plsc

N = 10000
E = 320000
NPAD = 10240
ROWS_PER_TILE = 640
CHW = 128
FC = 120
SCC = 40
BS = 40
TOTCH = 16 * (FC + SCC)
EPAD = TOTCH * CHW
RD = 8
RL = 2
BPB = BS // RD
EPS = 1e-5

_mesh = plsc.VectorSubcoreMesh(core_axis_name="c", subcore_axis_name="s")
_sc_params = pltpu.CompilerParams(use_tc_tiling_on_sc=False)


def _fill_rows(buf, val, d):
    v16 = jnp.full((16,), val, jnp.float32)

    def body(i, carry):
        for cc in range(d // 16):
            buf[i, pl.ds(cc * 16, 16)] = v16
        return carry

    lax.fori_loop(0, CHW, body, 0)


def _load_idx(c, s, idx_hbm, idx_v):
    @pl.when(c == 0)
    def _():
        pltpu.sync_copy(idx_hbm.at[pl.ds(s * FC, FC)], idx_v)

    @pl.when(c == 1)
    def _():
        pltpu.sync_copy(idx_hbm.at[pl.ds(16 * FC + s * SCC, SCC)],
                        idx_v.at[pl.ds(0, SCC)])


@functools.partial(
    pl.kernel,
    out_type=jax.ShapeDtypeStruct((2, NPAD, 16), jnp.float32),
    mesh=_mesh,
    compiler_params=_sc_params,
    scratch_types=[
        pltpu.VMEM((FC, CHW), jnp.int32),
        pltpu.VMEM((CHW, 16), jnp.float32),
        pltpu.VMEM_SHARED((NPAD, 16), jnp.float32),
        pltpu.SemaphoreType.DMA,
    ],
)
def _sc_degree(dst_hbm, out_hbm, didx, buf, degsh, sem):
    c = lax.axis_index("c")
    s = lax.axis_index("s")
    nch = jnp.where(c == 0, FC, SCC)

    _load_idx(c, s, dst_hbm, didx)
    _fill_rows(buf, 0.0, 16)
    for k in range(ROWS_PER_TILE // CHW):
        pltpu.sync_copy(buf, degsh.at[pl.ds(s * ROWS_PER_TILE + k * CHW, CHW)])
    _fill_rows(buf, 1.0, 16)
    plsc.subcore_barrier()

    def fire(j, carry):
        pltpu.async_copy(buf, degsh.at[didx.at[j]], sem, add=True)
        return carry

    def drain(j, carry):
        pltpu.make_async_copy(buf, degsh.at[didx.at[j]], sem).wait()
        return carry

    lax.fori_loop(0, nch, fire, 0)
    lax.fori_loop(0, nch, drain, 0)
    plsc.subcore_barrier()
    pltpu.sync_copy(
        degsh.at[pl.ds(s * ROWS_PER_TILE, ROWS_PER_TILE)],
        out_hbm.at[c].at[pl.ds(s * ROWS_PER_TILE, ROWS_PER_TILE)],
    )


def _make_sc_agg(d):
    @functools.partial(
        pl.kernel,
        out_type=jax.ShapeDtypeStruct((2, NPAD, d), jnp.float32),
        mesh=_mesh,
        compiler_params=_sc_params,
        scratch_types=[
            pltpu.VMEM((2, BS, CHW), jnp.int32),
            pltpu.VMEM((2, BS, CHW), jnp.int32),
            pltpu.VMEM((RD, CHW, d), jnp.float32),
            pltpu.VMEM_SHARED((NPAD, d), jnp.float32),
            pltpu.SemaphoreType.DMA,
            pltpu.SemaphoreType.DMA,
            pltpu.SemaphoreType.DMA,
        ],
    )
    def sc_agg(hp_hbm, src_hbm, dst_hbm, out_hbm, sidx, didx, rows,
               aggsh, gsem, ssem, isem):
        c = lax.axis_index("c")
        s = lax.axis_index("s")
        nch = jnp.where(c == 0, FC, SCC)
        ntrip = jnp.where(c == 0, FC // RD, SCC // RD)
        nblk = jnp.where(c == 0, FC // BS, SCC // BS)
        cbase = jnp.where(c == 0, s * FC, 16 * FC + s * SCC)

        def fire_idx(m):
            pltpu.async_copy(src_hbm.at[pl.ds(cbase + m * BS, BS)],
                             sidx.at[m % 2], isem)
            pltpu.async_copy(dst_hbm.at[pl.ds(cbase + m * BS, BS)],
                             didx.at[m % 2], isem)

        def wait_idx():
            pltpu.make_async_copy(src_hbm.at[pl.ds(cbase, BS)],
                                  sidx.at[0], isem).wait()
            pltpu.make_async_copy(dst_hbm.at[pl.ds(cbase, BS)],
                                  didx.at[0], isem).wait()

        pltpu.sync_copy(src_hbm.at[pl.ds(cbase, BS)], sidx.at[0])
        pltpu.sync_copy(dst_hbm.at[pl.ds(cbase, BS)], didx.at[0])

        _fill_rows(rows.at[0], 0.0, d)
        for k in range(ROWS_PER_TILE // CHW):
            pltpu.sync_copy(
                rows.at[0], aggsh.at[pl.ds(s * ROWS_PER_TILE + k * CHW, CHW)])
        plsc.subcore_barrier()

        def idx_row(arr, j):
            return arr.at[(j // BS) % 2].at[j % BS]

        def fire_g(b, j):
            pltpu.async_copy(hp_hbm.at[idx_row(sidx, j)], rows.at[b], gsem)

        def wait_g(b):
            pltpu.make_async_copy(
                hp_hbm.at[sidx.at[0].at[0]], rows.at[b], gsem).wait()

        def fire_s(b, j):
            pltpu.async_copy(rows.at[b], aggsh.at[idx_row(didx, j)], ssem,
                             add=True)

        def wait_s(b):
            pltpu.make_async_copy(
                rows.at[b], aggsh.at[didx.at[0].at[0]], ssem).wait()

        for j in range(RD - RL):
            fire_g(j, j)

        def body(t2, carry):
            m = t2 // BPB
            phase = t2 % BPB
            for b in range(RD):
                t = t2 * RD + b

                @pl.when(t >= RL)
                def _():
                    wait_s((t - RL) % RD)

                if b == RL:
                    @pl.when((phase == 0) & (m + 1 < nblk))
                    def _():
                        fire_idx(m + 1)

                    @pl.when((phase == BPB - 1) & (m + 1 < nblk))
                    def _():
                        wait_idx()

                @pl.when(t < nch - RD + RL)
                def _():
                    fire_g((t - RL) % RD, t - RL + RD)

                wait_g(b)
                fire_s(b, t)
            return carry

        lax.fori_loop(0, ntrip, body, 0)
        for i in range(RL):
            wait_s(RD - RL + i)
        plsc.subcore_barrier()
        pltpu.sync_copy(
            aggsh.at[pl.ds(s * ROWS_PER_TILE, ROWS_PER_TILE)],
            out_hbm.at[c].at[pl.ds(s * ROWS_PER_TILE, ROWS_PER_TILE)],
        )

    return sc_agg


_sc_agg64 = _make_sc_agg(64)
_sc_agg16 = _make_sc_agg(16)


def _dinv_from(degp_ref):
    deg = degp_ref[0, :N, 0:1] + degp_ref[1, :N, 0:1] + 1.0
    return lax.rsqrt(deg)


def _tc_mm(x, w1):
    def body(x_ref, w_ref, out_ref):
        out_ref[...] = jnp.dot(x_ref[...], w_ref[...],
                               preferred_element_type=jnp.float32)

    return pl.pallas_call(
        body, out_shape=jax.ShapeDtypeStruct((N, w1.shape[1]), jnp.float32),
    )(x, w1)


def _tc_scale(xw, degp):
    def body(xw_ref, degp_ref, out_ref):
        out_ref[...] = xw_ref[...] * _dinv_from(degp_ref)

    return pl.pallas_call(
        body, out_shape=jax.ShapeDtypeStruct(xw.shape, jnp.float32),
    )(xw, degp)


def _tc_mid(aggp, hp, degp, g, be, wn):
    def body(aggp_ref, hp_ref, degp_ref, g_ref, be_ref, w_ref, out_ref):
        dinv = _dinv_from(degp_ref)
        conv = dinv * (aggp_ref[0, :N] + aggp_ref[1, :N] + hp_ref[...])
        mu = jnp.mean(conv, axis=0, keepdims=True)
        xc = conv - mu
        var = jnp.mean(xc * xc, axis=0, keepdims=True)
        h = g_ref[...] * xc * lax.rsqrt(var + EPS) + be_ref[...]
        h = jnp.maximum(h, 0.0) * dinv
        out_ref[...] = jnp.dot(h, w_ref[...],
                               preferred_element_type=jnp.float32)

    return pl.pallas_call(
        body, out_shape=jax.ShapeDtypeStruct((N, wn.shape[1]), jnp.float32),
    )(aggp, hp, degp, g, be, wn)


def _tc_post(aggp, hp, degp, g, be):
    def body(aggp_ref, hp_ref, degp_ref, g_ref, be_ref, out_ref):
        dinv = _dinv_from(degp_ref)
        conv = dinv * (aggp_ref[0, :N] + aggp_ref[1, :N] + hp_ref[...])
        mu = jnp.mean(conv, axis=0, keepdims=True)
        xc = conv - mu
        var = jnp.mean(xc * xc, axis=0, keepdims=True)
        out_ref[...] = g_ref[...] * xc * lax.rsqrt(var + EPS) + be_ref[...]

    return pl.pallas_call(
        body, out_shape=jax.ShapeDtypeStruct((N, 16), jnp.float32),
    )(aggp, hp, degp, g, be)


def kernel(x, edge_index, W1, b1, g1, be1, W2, b2, g2, be2, W3, b3, g3, be3):
    del b1, b2, b3
    src = edge_index[0].astype(jnp.int32)
    dst = edge_index[1].astype(jnp.int32)
    pad = EPAD - E
    src_t = jnp.concatenate([src, jnp.zeros((pad,), jnp.int32)]
                            ).reshape(TOTCH, CHW)
    dst_t = jnp.concatenate([dst, jnp.full((pad,), N, jnp.int32)]
                            ).reshape(TOTCH, CHW)

    degp = _sc_degree(dst_t)

    g1r = g1.reshape(1, -1)
    be1r = be1.reshape(1, -1)
    g2r = g2.reshape(1, -1)
    be2r = be2.reshape(1, -1)
    g3p = jnp.concatenate([g3, jnp.ones((16 - g3.shape[0],), jnp.float32)]
                          ).reshape(1, 16)
    be3p = jnp.concatenate([be3, jnp.zeros((16 - be3.shape[0],), jnp.float32)]
                           ).reshape(1, 16)
    w3p = jnp.concatenate(
        [W3, jnp.zeros((W3.shape[0], 16 - W3.shape[1]), jnp.float32)], axis=1)

    hp1 = _tc_scale(_tc_mm(x, W1), degp)
    agg1 = _sc_agg64(hp1, src_t, dst_t)
    hp2 = _tc_mid(agg1, hp1, degp, g1r, be1r, W2)
    agg2 = _sc_agg64(hp2, src_t, dst_t)
    hp3 = _tc_mid(agg2, hp2, degp, g2r, be2r, w3p)
    agg3 = _sc_agg16(hp3, src_t, dst_t)
    out = _tc_post(agg3, hp3, degp, g3p, be3p)
    return out[:, :W3.shape[1]]


# speedup vs baseline: 1.5541x; 1.0812x over previous
import functools

import jax
import jax.numpy as jnp
from jax import lax
from jax.experimental import pallas as pl
from jax.experimental.pallas import tpu as pltpu
from jax.experimental.pallas import tpu_sc as plsc

N = 10000
E = 320000
NPAD = 10240
ROWS_PER_TILE = 640
CHW = 128
FC = 144
SCC = 16
BS = 16
TOTCH = 16 * (FC + SCC)
EPAD = TOTCH * CHW
RD = 8
RL = 2
BPB = BS // RD
EPS = 1e-5

_mesh = plsc.VectorSubcoreMesh(core_axis_name="c", subcore_axis_name="s")
_sc_params = pltpu.CompilerParams(use_tc_tiling_on_sc=False)


def _fill_rows(buf, val, d):
    v16 = jnp.full((16,), val, jnp.float32)

    def body(i, carry):
        for cc in range(d // 16):
            buf[i, pl.ds(cc * 16, 16)] = v16
        return carry

    lax.fori_loop(0, CHW, body, 0)


def _load_idx(c, s, idx_hbm, idx_v):
    @pl.when(c == 0)
    def _():
        pltpu.sync_copy(idx_hbm.at[pl.ds(s * FC, FC)], idx_v)

    @pl.when(c == 1)
    def _():
        pltpu.sync_copy(idx_hbm.at[pl.ds(16 * FC + s * SCC, SCC)],
                        idx_v.at[pl.ds(0, SCC)])


@functools.partial(
    pl.kernel,
    out_type=jax.ShapeDtypeStruct((2, NPAD, 16), jnp.float32),
    mesh=_mesh,
    compiler_params=_sc_params,
    scratch_types=[
        pltpu.VMEM((FC, CHW), jnp.int32),
        pltpu.VMEM((CHW, 16), jnp.float32),
        pltpu.VMEM_SHARED((NPAD, 16), jnp.float32),
        pltpu.SemaphoreType.DMA,
    ],
)
def _sc_degree(dst_hbm, out_hbm, didx, buf, degsh, sem):
    c = lax.axis_index("c")
    s = lax.axis_index("s")
    nch = jnp.where(c == 0, FC, SCC)

    _load_idx(c, s, dst_hbm, didx)
    _fill_rows(buf, 0.0, 16)
    for k in range(ROWS_PER_TILE // CHW):
        pltpu.sync_copy(buf, degsh.at[pl.ds(s * ROWS_PER_TILE + k * CHW, CHW)])
    _fill_rows(buf, 1.0, 16)
    plsc.subcore_barrier()

    def fire(j, carry):
        pltpu.async_copy(buf, degsh.at[didx.at[j]], sem, add=True)
        return carry

    def drain(j, carry):
        pltpu.make_async_copy(buf, degsh.at[didx.at[j]], sem).wait()
        return carry

    lax.fori_loop(0, nch, fire, 0)
    lax.fori_loop(0, nch, drain, 0)
    plsc.subcore_barrier()
    pltpu.sync_copy(
        degsh.at[pl.ds(s * ROWS_PER_TILE, ROWS_PER_TILE)],
        out_hbm.at[c].at[pl.ds(s * ROWS_PER_TILE, ROWS_PER_TILE)],
    )


def _make_sc_agg(d):
    @functools.partial(
        pl.kernel,
        out_type=jax.ShapeDtypeStruct((2, NPAD, d), jnp.float32),
        mesh=_mesh,
        compiler_params=_sc_params,
        scratch_types=[
            pltpu.VMEM((2, BS, CHW), jnp.int32),
            pltpu.VMEM((2, BS, CHW), jnp.int32),
            pltpu.VMEM((RD, CHW, d), jnp.float32),
            pltpu.VMEM_SHARED((NPAD, d), jnp.float32),
            pltpu.SemaphoreType.DMA,
            pltpu.SemaphoreType.DMA,
            pltpu.SemaphoreType.DMA,
        ],
    )
    def sc_agg(hp_hbm, src_hbm, dst_hbm, out_hbm, sidx, didx, rows,
               aggsh, gsem, ssem, isem):
        c = lax.axis_index("c")
        s = lax.axis_index("s")
        nch = jnp.where(c == 0, FC, SCC)
        ntrip = jnp.where(c == 0, FC // RD, SCC // RD)
        nblk = jnp.where(c == 0, FC // BS, SCC // BS)
        cbase = jnp.where(c == 0, s * FC, 16 * FC + s * SCC)

        def fire_idx(m):
            pltpu.async_copy(src_hbm.at[pl.ds(cbase + m * BS, BS)],
                             sidx.at[m % 2], isem)
            pltpu.async_copy(dst_hbm.at[pl.ds(cbase + m * BS, BS)],
                             didx.at[m % 2], isem)

        def wait_idx():
            pltpu.make_async_copy(src_hbm.at[pl.ds(cbase, BS)],
                                  sidx.at[0], isem).wait()
            pltpu.make_async_copy(dst_hbm.at[pl.ds(cbase, BS)],
                                  didx.at[0], isem).wait()

        pltpu.sync_copy(src_hbm.at[pl.ds(cbase, BS)], sidx.at[0])
        pltpu.sync_copy(dst_hbm.at[pl.ds(cbase, BS)], didx.at[0])

        _fill_rows(rows.at[0], 0.0, d)
        for k in range(ROWS_PER_TILE // CHW):
            pltpu.sync_copy(
                rows.at[0], aggsh.at[pl.ds(s * ROWS_PER_TILE + k * CHW, CHW)])
        plsc.subcore_barrier()

        def idx_row(arr, j):
            return arr.at[(j // BS) % 2].at[j % BS]

        def fire_g(b, j):
            pltpu.async_copy(hp_hbm.at[idx_row(sidx, j)], rows.at[b], gsem)

        def wait_g(b):
            pltpu.make_async_copy(
                hp_hbm.at[sidx.at[0].at[0]], rows.at[b], gsem).wait()

        def fire_s(b, j):
            pltpu.async_copy(rows.at[b], aggsh.at[idx_row(didx, j)], ssem,
                             add=True)

        def wait_s(b):
            pltpu.make_async_copy(
                rows.at[b], aggsh.at[didx.at[0].at[0]], ssem).wait()

        for j in range(RD - RL):
            fire_g(j, j)

        def body(t2, carry):
            m = t2 // BPB
            phase = t2 % BPB
            for b in range(RD):
                t = t2 * RD + b

                @pl.when(t >= RL)
                def _():
                    wait_s((t - RL) % RD)

                if b == RL:
                    @pl.when((phase == 0) & (m + 1 < nblk))
                    def _():
                        fire_idx(m + 1)

                    @pl.when((phase == BPB - 1) & (m + 1 < nblk))
                    def _():
                        wait_idx()

                @pl.when(t < nch - RD + RL)
                def _():
                    fire_g((t - RL) % RD, t - RL + RD)

                wait_g(b)
                fire_s(b, t)
            return carry

        lax.fori_loop(0, ntrip, body, 0)
        for i in range(RL):
            wait_s(RD - RL + i)
        plsc.subcore_barrier()
        pltpu.sync_copy(
            aggsh.at[pl.ds(s * ROWS_PER_TILE, ROWS_PER_TILE)],
            out_hbm.at[c].at[pl.ds(s * ROWS_PER_TILE, ROWS_PER_TILE)],
        )

    return sc_agg


_sc_agg64 = _make_sc_agg(64)
_sc_agg16 = _make_sc_agg(16)


def _dinv_from(degp_ref):
    deg = degp_ref[0, :N, 0:1] + degp_ref[1, :N, 0:1] + 1.0
    return lax.rsqrt(deg)


def _tc_mm(x, w1):
    def body(x_ref, w_ref, out_ref):
        out_ref[...] = jnp.dot(x_ref[...], w_ref[...],
                               preferred_element_type=jnp.float32)

    return pl.pallas_call(
        body, out_shape=jax.ShapeDtypeStruct((N, w1.shape[1]), jnp.float32),
    )(x, w1)


def _tc_scale(xw, degp):
    def body(xw_ref, degp_ref, out_ref):
        out_ref[...] = xw_ref[...] * _dinv_from(degp_ref)

    return pl.pallas_call(
        body, out_shape=jax.ShapeDtypeStruct(xw.shape, jnp.float32),
    )(xw, degp)


def _tc_mid(aggp, hp, degp, g, be, wn):
    def body(aggp_ref, hp_ref, degp_ref, g_ref, be_ref, w_ref, out_ref):
        dinv = _dinv_from(degp_ref)
        conv = dinv * (aggp_ref[0, :N] + aggp_ref[1, :N] + hp_ref[...])
        mu = jnp.mean(conv, axis=0, keepdims=True)
        xc = conv - mu
        var = jnp.mean(xc * xc, axis=0, keepdims=True)
        h = g_ref[...] * xc * lax.rsqrt(var + EPS) + be_ref[...]
        h = jnp.maximum(h, 0.0) * dinv
        out_ref[...] = jnp.dot(h, w_ref[...],
                               preferred_element_type=jnp.float32)

    return pl.pallas_call(
        body, out_shape=jax.ShapeDtypeStruct((N, wn.shape[1]), jnp.float32),
    )(aggp, hp, degp, g, be, wn)


def _tc_post(aggp, hp, degp, g, be):
    def body(aggp_ref, hp_ref, degp_ref, g_ref, be_ref, out_ref):
        dinv = _dinv_from(degp_ref)
        conv = dinv * (aggp_ref[0, :N] + aggp_ref[1, :N] + hp_ref[...])
        mu = jnp.mean(conv, axis=0, keepdims=True)
        xc = conv - mu
        var = jnp.mean(xc * xc, axis=0, keepdims=True)
        out_ref[...] = g_ref[...] * xc * lax.rsqrt(var + EPS) + be_ref[...]

    return pl.pallas_call(
        body, out_shape=jax.ShapeDtypeStruct((N, 16), jnp.float32),
    )(aggp, hp, degp, g, be)


def kernel(x, edge_index, W1, b1, g1, be1, W2, b2, g2, be2, W3, b3, g3, be3):
    del b1, b2, b3
    src = edge_index[0].astype(jnp.int32)
    dst = edge_index[1].astype(jnp.int32)
    pad = EPAD - E
    src_t = jnp.concatenate([src, jnp.zeros((pad,), jnp.int32)]
                            ).reshape(TOTCH, CHW)
    dst_t = jnp.concatenate([dst, jnp.full((pad,), N, jnp.int32)]
                            ).reshape(TOTCH, CHW)

    degp = _sc_degree(dst_t)

    g1r = g1.reshape(1, -1)
    be1r = be1.reshape(1, -1)
    g2r = g2.reshape(1, -1)
    be2r = be2.reshape(1, -1)
    g3p = jnp.concatenate([g3, jnp.ones((16 - g3.shape[0],), jnp.float32)]
                          ).reshape(1, 16)
    be3p = jnp.concatenate([be3, jnp.zeros((16 - be3.shape[0],), jnp.float32)]
                           ).reshape(1, 16)
    w3p = jnp.concatenate(
        [W3, jnp.zeros((W3.shape[0], 16 - W3.shape[1]), jnp.float32)], axis=1)

    hp1 = _tc_scale(_tc_mm(x, W1), degp)
    agg1 = _sc_agg64(hp1, src_t, dst_t)
    hp2 = _tc_mid(agg1, hp1, degp, g1r, be1r, W2)
    agg2 = _sc_agg64(hp2, src_t, dst_t)
    hp3 = _tc_mid(agg2, hp2, degp, g2r, be2r, w3p)
    agg3 = _sc_agg16(hp3, src_t, dst_t)
    out = _tc_post(agg3, hp3, degp, g3p, be3p)
    return out[:, :W3.shape[1]]
